# baseline (speedup 1.0000x reference)
import jax
from jax.experimental import pallas as pl
from jax.experimental.pallas import tpu as pltpu


def _identity_copy_kernel(x_ref, rel_ref, x_out_ref, rel_out_ref, sem_x, sem_r):
    copy_x = pltpu.make_async_copy(x_ref, x_out_ref, sem_x)
    copy_r = pltpu.make_async_copy(rel_ref, rel_out_ref, sem_r)
    copy_x.start()
    copy_r.start()
    copy_x.wait()
    copy_r.wait()


def kernel(x, edge_index, edge_type, rel_embed):
    x_out, rel_out = pl.pallas_call(
        _identity_copy_kernel,
        in_specs=[
            pl.BlockSpec(memory_space=pl.MemorySpace.ANY),
            pl.BlockSpec(memory_space=pl.MemorySpace.ANY),
        ],
        out_specs=[
            pl.BlockSpec(memory_space=pl.MemorySpace.ANY),
            pl.BlockSpec(memory_space=pl.MemorySpace.ANY),
        ],
        out_shape=[
            jax.ShapeDtypeStruct(x.shape, x.dtype),
            jax.ShapeDtypeStruct(rel_embed.shape, rel_embed.dtype),
        ],
        scratch_shapes=[pltpu.SemaphoreType.DMA, pltpu.SemaphoreType.DMA],
    )(x, rel_embed)
    return (x_out, rel_out)
